# Initial kernel scaffold; baseline (speedup 1.0000x reference)
import jax
import jax.numpy as jnp
from jax.experimental import pallas as pl


def kernel(x, W_enc, b_enc, W_dec, b_dec):
    raise NotImplementedError("write your pallas kernel here")


# speedup vs baseline: 1.8098x; 1.8098x over previous
import functools

import jax
import jax.numpy as jnp
from jax import lax
from jax.experimental import pallas as pl
from jax.experimental.pallas import tpu as pltpu
from jax.experimental.pallas import tpu_sc as plsc

D_IN = 1024
HID = 32768
K = 32
B = 64
L = 16
CHUNKS = HID // L
NBUCKET = 1024
BSHIFT = 21
CAP = 256
CANDBUF = CAP + 2 * L
NSC = 2
NSUB = 16


def _enc_body(x_ref, w_ref, benc_ref, bdec_ref, o_ref):
    sae = x_ref[...] - bdec_ref[...]
    acts = lax.dot_general(sae, w_ref[...], (((1,), (1,)), ((), ())),
                           preferred_element_type=jnp.float32)
    o_ref[...] = jnp.maximum(acts + benc_ref[...], 0.0)


def _encode(x, W_enc, b_enc2d, b_dec2d):
    BH = 2048
    return pl.pallas_call(
        _enc_body,
        grid=(HID // BH,),
        in_specs=[
            pl.BlockSpec((B, D_IN), lambda j: (0, 0)),
            pl.BlockSpec((BH, D_IN), lambda j: (j, 0)),
            pl.BlockSpec((1, BH), lambda j: (0, j)),
            pl.BlockSpec((1, D_IN), lambda j: (0, 0)),
        ],
        out_specs=pl.BlockSpec((B, BH), lambda j: (0, j)),
        out_shape=jax.ShapeDtypeStruct((B, HID), jnp.float32),
    )(x, W_enc, b_enc2d, b_dec2d)


def _sc_body(pre_hbm, wdec_hbm, bdec_hbm, out_hbm,
             row_v, hist_v, cand_v, candi_v, topv_s, topi_v,
             rows_v, acc_v, bdec_v, sem):
    wid = lax.axis_index("s") * NSC + lax.axis_index("c")
    lane = lax.iota(jnp.int32, L)
    ones_i = jnp.ones((L,), jnp.int32)
    zeros_i = jnp.zeros((L,), jnp.int32)

    pltpu.sync_copy(bdec_hbm, bdec_v)

    for r in range(2):
        row_id = wid * 2 + r
        pltpu.sync_copy(pre_hbm.at[row_id], row_v)

        def zbody(i, _):
            hist_v[pl.ds(i * L, L)] = zeros_i
            return 0
        lax.fori_loop(0, NBUCKET, zbody, 0)

        def hbody(j, bmaxv):
            v = row_v[pl.ds(j * L, L)]
            key = plsc.bitcast(v, jnp.uint32)
            b = jnp.minimum(key >> BSHIFT, jnp.uint32(NBUCKET - 1))
            bi = b.astype(jnp.int32)
            plsc.addupdate_scatter(hist_v, [bi * L + lane], ones_i)
            return jnp.maximum(bmaxv, bi)
        bmaxv = lax.fori_loop(0, CHUNKS, hbody, zeros_i)
        bmax = jnp.max(bmaxv)

        def wcond(s):
            bb, n, c = s
            return jnp.logical_and(n + c < K, bb > 0)

        def wbody(s):
            bb, n, c = s
            b2 = bb - 1
            c2 = jnp.sum(hist_v[pl.ds(b2 * L, L)])
            return (b2, n + c, c2)

        c0 = jnp.sum(hist_v[pl.ds(bmax * L, L)])
        bstar, _, _ = lax.while_loop(wcond, wbody,
                                     (bmax, jnp.int32(0), c0))

        tvec = plsc.bitcast(jnp.full((L,), bstar, jnp.int32) << BSHIFT,
                            jnp.float32)

        neg1 = jnp.full((L,), -1.0, jnp.float32)

        def cinit(i, _):
            cand_v[pl.ds(i * L, L)] = neg1
            return 0
        lax.fori_loop(0, CANDBUF // L, cinit, 0)

        def sbody(j, cnt):
            v = row_v[pl.ds(j * L, L)]
            m = v >= tvec
            c = plsc.all_reduce_population_count(m)[0]
            plsc.store_compressed(cand_v.at[pl.ds(cnt, L)], v, mask=m)
            plsc.store_compressed(candi_v.at[pl.ds(cnt, L)],
                                  j * L + lane, mask=m)
            return jnp.minimum(cnt + c, CAP)
        lax.fori_loop(0, CHUNKS, sbody, jnp.int32(0))

        bigi = jnp.full((L,), jnp.int32(1 << 30), jnp.int32)
        lane0 = lane == 0
        nvec = CANDBUF // L

        def ebody(kk, _):
            def mbody(i, mv):
                return jnp.maximum(mv, cand_v[pl.ds(i * L, L)])
            mv = lax.fori_loop(0, nvec, mbody,
                               jnp.full((L,), -2.0, jnp.float32))
            gm = jnp.max(mv)
            gms = jnp.full((L,), gm, jnp.float32)

            def ibody(i, iv):
                cv = cand_v[pl.ds(i * L, L)]
                ci = candi_v[pl.ds(i * L, L)]
                return jnp.minimum(iv, jnp.where(cv == gms, ci, bigi))
            iv = lax.fori_loop(0, nvec, ibody, bigi)
            mis = jnp.full((L,), jnp.min(iv), jnp.int32)

            def rbody(i, _):
                cv = cand_v[pl.ds(i * L, L)]
                ci = candi_v[pl.ds(i * L, L)]
                hit = jnp.logical_and(cv == gms, ci == mis)
                cand_v[pl.ds(i * L, L)] = jnp.where(hit, -1.0, cv)
                return 0
            lax.fori_loop(0, nvec, rbody, 0)

            kks = jnp.full((L,), kk, jnp.int32)
            topv_s[kk] = gm
            plsc.store_scatter(topi_v, [kks],
                               jnp.minimum(mis, jnp.int32(HID - 1)),
                               mask=lane0)
            return 0
        lax.fori_loop(0, K, ebody, 0)

        pltpu.async_copy(wdec_hbm.at[topi_v], rows_v, sem).wait()

        def abody(c, _):
            acc_v[pl.ds(c * L, L)] = bdec_v[pl.ds(c * L, L)]
            return 0
        lax.fori_loop(0, D_IN // L, abody, 0)

        for kb in range(K // 8):
            bcs = [jnp.full((L,), topv_s[kb * 8 + u], jnp.float32)
                   for u in range(8)]

            def dbody(c, _, kb=kb, bcs=bcs):
                a = acc_v[pl.ds(c * L, L)]
                for u in range(8):
                    a = a + bcs[u] * rows_v[kb * 8 + u, pl.ds(c * L, L)]
                acc_v[pl.ds(c * L, L)] = a
                return 0
            lax.fori_loop(0, D_IN // L, dbody, 0)

        pltpu.sync_copy(acc_v, out_hbm.at[row_id])


@functools.partial(
    pl.kernel,
    out_type=jax.ShapeDtypeStruct((B, D_IN), jnp.float32),
    mesh=plsc.VectorSubcoreMesh(core_axis_name="c", subcore_axis_name="s",
                                num_cores=NSC, num_subcores=NSUB),
    compiler_params=pltpu.CompilerParams(needs_layout_passes=False),
    scratch_types=[
        pltpu.VMEM((HID,), jnp.float32),
        pltpu.VMEM((NBUCKET * L,), jnp.int32),
        pltpu.VMEM((CANDBUF,), jnp.float32),
        pltpu.VMEM((CANDBUF,), jnp.int32),
        pltpu.SMEM((K,), jnp.float32),
        pltpu.VMEM((K,), jnp.int32),
        pltpu.VMEM((K, D_IN), jnp.float32),
        pltpu.VMEM((D_IN,), jnp.float32),
        pltpu.VMEM((D_IN,), jnp.float32),
        pltpu.SemaphoreType.DMA,
    ],
)
def _sc_topk_decode(pre_hbm, wdec_hbm, bdec_hbm, out_hbm, *scratch):
    _sc_body(pre_hbm, wdec_hbm, bdec_hbm, out_hbm, *scratch)


def kernel(x, W_enc, b_enc, W_dec, b_dec):
    pre = _encode(x, W_enc, b_enc.reshape(1, HID), b_dec.reshape(1, D_IN))
    return _sc_topk_decode(pre, W_dec, b_dec)


# speedup vs baseline: 2.1060x; 1.1637x over previous
import functools

import jax
import jax.numpy as jnp
from jax import lax
from jax.experimental import pallas as pl
from jax.experimental.pallas import tpu as pltpu
from jax.experimental.pallas import tpu_sc as plsc

D_IN = 1024
HID = 32768
K = 32
B = 64
L = 16
CHUNKS = HID // L
NBUCKET = 1024
BSHIFT = 21
SLOT = 32
EQN = 4
NSC = 2
NSUB = 16


def _enc_body(x_ref, w_ref, benc_ref, bdec_ref, o_ref):
    sae = x_ref[...] - bdec_ref[...]
    acts = lax.dot_general(sae, w_ref[...], (((1,), (1,)), ((), ())),
                           preferred_element_type=jnp.float32)
    o_ref[...] = jnp.maximum(acts + benc_ref[...], 0.0)


def _encode(x, W_enc, b_enc2d, b_dec2d):
    BH = 2048
    return pl.pallas_call(
        _enc_body,
        grid=(HID // BH,),
        in_specs=[
            pl.BlockSpec((B, D_IN), lambda j: (0, 0)),
            pl.BlockSpec((BH, D_IN), lambda j: (j, 0)),
            pl.BlockSpec((1, BH), lambda j: (0, j)),
            pl.BlockSpec((1, D_IN), lambda j: (0, 0)),
        ],
        out_specs=pl.BlockSpec((B, BH), lambda j: (0, j)),
        out_shape=jax.ShapeDtypeStruct((B, HID), jnp.float32),
    )(x, W_enc, b_enc2d, b_dec2d)


def _sc_body(pre_hbm, wdec_hbm, bdec_hbm, out_hbm,
             row_v, hist_v, cand_v, candi_v, topv_v, topi_v, topi32_v,
             eqi_v, bc_v, rows_v, acc_v, bdec_v, sem):
    wid = lax.axis_index("s") * NSC + lax.axis_index("c")
    lane = lax.iota(jnp.int32, L)
    ones_i = jnp.ones((L,), jnp.int32)
    zeros_i = jnp.zeros((L,), jnp.int32)
    bigi = jnp.full((L,), jnp.int32(1 << 30), jnp.int32)
    lane0 = lane == 0

    pltpu.sync_copy(bdec_hbm, bdec_v)

    for r in range(2):
        row_id = wid * 2 + r
        pltpu.sync_copy(pre_hbm.at[row_id], row_v)

        def zbody(i, _):
            for u in range(8):
                hist_v[pl.ds((i * 8 + u) * L, L)] = zeros_i
            return 0
        lax.fori_loop(0, NBUCKET // 8, zbody, 0)

        def hbody(jj, am):
            for u in range(4):
                j = jj * 4 + u
                v = row_v[pl.ds(j * L, L)]
                key = plsc.bitcast(v, jnp.uint32) & jnp.uint32(0x7FFFFFFF)
                addr = ((key >> jnp.uint32(BSHIFT - 4)) &
                        jnp.uint32(0xFFF0)).astype(jnp.int32) | lane
                plsc.addupdate_scatter(hist_v, [addr], ones_i)
                am = jnp.maximum(am, addr)
            return am
        amax = lax.fori_loop(0, CHUNKS // 4, hbody, zeros_i)
        bmax = jnp.max(amax) >> 4

        def wcond(s):
            bb, n, c = s
            return jnp.logical_and(n + c < K, bb > 0)

        def wbody(s):
            bb, n, c = s
            b2 = bb - 1
            c2 = jnp.sum(hist_v[pl.ds(b2 * L, L)])
            return (b2, n + c, c2)

        c0 = jnp.sum(hist_v[pl.ds(bmax * L, L)])
        bstar, _, _ = lax.while_loop(wcond, wbody,
                                     (bmax, jnp.int32(0), c0))
        tkey_lo = bstar << BSHIFT
        tvec = plsc.bitcast(jnp.full((L,), tkey_lo, jnp.int32), jnp.float32)

        neg1 = jnp.full((L,), -1.0, jnp.float32)

        def cinit(i, _):
            for u in range(4):
                cand_v[pl.ds((i * 4 + u) * L, L)] = neg1
            return 0
        lax.fori_loop(0, SLOT // 4, cinit, 0)

        def sbody(jj, cnt_v):
            for u in range(2):
                j = jj * 2 + u
                v = row_v[pl.ds(j * L, L)]
                m = v >= tvec
                addr = jnp.minimum(cnt_v, SLOT - 1) * L + lane
                plsc.store_scatter(cand_v, [addr], v, mask=m)
                plsc.store_scatter(candi_v, [addr], j * L + lane, mask=m)
                cnt_v = cnt_v + jnp.where(m, 1, 0)
            return cnt_v
        lax.fori_loop(0, CHUNKS // 2, sbody, zeros_i)

        hi0 = ((bstar + 1) << BSHIFT) - 1

        def bsbody(it, st):
            lo, hi = st
            mid = lo + ((hi - lo + 1) >> 1)
            tm = plsc.bitcast(jnp.full((L,), mid, jnp.int32), jnp.float32)

            def cb(i, cv):
                return cv + jnp.where(cand_v[pl.ds(i * L, L)] >= tm, 1, 0)
            c = jnp.sum(lax.fori_loop(0, SLOT, cb, zeros_i))
            ge = c >= K
            return (jnp.where(ge, mid, lo), jnp.where(ge, hi, mid - 1))
        tstar, _ = lax.fori_loop(0, BSHIFT, bsbody, (tkey_lo, hi0))
        tsv = plsc.bitcast(jnp.full((L,), tstar, jnp.int32), jnp.float32)

        def gcomp(i, cnt):
            cv_ = cand_v[pl.ds(i * L, L)]
            ci_ = candi_v[pl.ds(i * L, L)]
            m = cv_ > tsv
            off = jnp.minimum(cnt, K)
            plsc.store_compressed(topv_v.at[pl.ds(off, L)], cv_, mask=m)
            plsc.store_compressed(topi_v.at[pl.ds(off, L)], ci_, mask=m)
            return cnt + plsc.all_reduce_population_count(m)[0]
        cnt_gt = jnp.minimum(lax.fori_loop(0, SLOT, gcomp, jnp.int32(0)),
                             jnp.int32(K - 1))

        for i in range(EQN):
            eqi_v[pl.ds(i * L, L)] = bigi

        def ecomp(i, cnt):
            cv_ = cand_v[pl.ds(i * L, L)]
            ci_ = candi_v[pl.ds(i * L, L)]
            m = cv_ == tsv
            off = jnp.minimum(cnt, (EQN - 1) * L)
            plsc.store_compressed(eqi_v.at[pl.ds(off, L)], ci_, mask=m)
            return cnt + plsc.all_reduce_population_count(m)[0]
        lax.fori_loop(0, SLOT, ecomp, jnp.int32(0))

        def rbody(rr, _):
            def mb(i, mv):
                return jnp.minimum(mv, eqi_v[pl.ds(i * L, L)])
            mi = jnp.min(lax.fori_loop(0, EQN, mb, bigi))
            mis = jnp.full((L,), mi, jnp.int32)

            def rm(i, _):
                ci_ = eqi_v[pl.ds(i * L, L)]
                eqi_v[pl.ds(i * L, L)] = jnp.where(ci_ == mis, bigi, ci_)
                return 0
            lax.fori_loop(0, EQN, rm, 0)
            ks = jnp.full((L,), cnt_gt + rr, jnp.int32)
            plsc.store_scatter(topi_v, [ks],
                               jnp.minimum(mis, jnp.int32(HID - 1)),
                               mask=lane0)
            plsc.store_scatter(topv_v, [ks], tsv, mask=lane0)
            return 0
        lax.fori_loop(0, K - cnt_gt, rbody, 0)

        for g in range(2):
            topi32_v[pl.ds(g * L, L)] = topi_v[pl.ds(g * L, L)]
            va = topv_v[pl.ds(g * L, L)]
            for j in range(L):
                addr = lane * L + g * (L * L) + ((lane + j) & (L - 1))
                plsc.store_scatter(bc_v, [addr], va)

        pltpu.async_copy(wdec_hbm.at[topi32_v], rows_v, sem).wait()

        def abody(c, _):
            acc_v[pl.ds(c * L, L)] = bdec_v[pl.ds(c * L, L)]
            return 0
        lax.fori_loop(0, D_IN // L, abody, 0)

        for kb in range(K // 8):
            bcs = [bc_v[pl.ds((kb * 8 + u) * L, L)] for u in range(8)]

            def dbody(c, _, kb=kb, bcs=bcs):
                a = acc_v[pl.ds(c * L, L)]
                for u in range(8):
                    a = a + bcs[u] * rows_v[kb * 8 + u, pl.ds(c * L, L)]
                acc_v[pl.ds(c * L, L)] = a
                return 0
            lax.fori_loop(0, D_IN // L, dbody, 0)

        pltpu.sync_copy(acc_v, out_hbm.at[row_id])


@functools.partial(
    pl.kernel,
    out_type=jax.ShapeDtypeStruct((B, D_IN), jnp.float32),
    mesh=plsc.VectorSubcoreMesh(core_axis_name="c", subcore_axis_name="s",
                                num_cores=NSC, num_subcores=NSUB),
    compiler_params=pltpu.CompilerParams(needs_layout_passes=False),
    scratch_types=[
        pltpu.VMEM((HID,), jnp.float32),
        pltpu.VMEM((NBUCKET * L,), jnp.int32),
        pltpu.VMEM((SLOT * L,), jnp.float32),
        pltpu.VMEM((SLOT * L,), jnp.int32),
        pltpu.VMEM((K + L,), jnp.float32),
        pltpu.VMEM((K + L,), jnp.int32),
        pltpu.VMEM((K,), jnp.int32),
        pltpu.VMEM((EQN * L,), jnp.int32),
        pltpu.VMEM((K * L,), jnp.float32),
        pltpu.VMEM((K, D_IN), jnp.float32),
        pltpu.VMEM((D_IN,), jnp.float32),
        pltpu.VMEM((D_IN,), jnp.float32),
        pltpu.SemaphoreType.DMA,
    ],
)
def _sc_topk_decode(pre_hbm, wdec_hbm, bdec_hbm, out_hbm, *scratch):
    _sc_body(pre_hbm, wdec_hbm, bdec_hbm, out_hbm, *scratch)


def kernel(x, W_enc, b_enc, W_dec, b_dec):
    pre = _encode(x, W_enc, b_enc.reshape(1, HID), b_dec.reshape(1, D_IN))
    return _sc_topk_decode(pre, W_dec, b_dec)


# speedup vs baseline: 2.6023x; 1.2356x over previous
import functools

import jax
import jax.numpy as jnp
from jax import lax
from jax.experimental import pallas as pl
from jax.experimental.pallas import tpu as pltpu
from jax.experimental.pallas import tpu_sc as plsc

D_IN = 1024
HID = 32768
K = 32
B = 64
L = 16
CHUNKS = HID // L
NBUCKET = 1024
BSHIFT = 21
SLOT = 32
EQN = 4
NSC = 2
NSUB = 16


def _enc_body(x_ref, w_ref, benc_ref, bdec_ref, o_ref):
    sae = x_ref[...] - bdec_ref[...]
    acts = lax.dot_general(sae, w_ref[...], (((1,), (1,)), ((), ())),
                           preferred_element_type=jnp.float32)
    o_ref[...] = jnp.maximum(acts + benc_ref[...], 0.0)


def _encode(x, W_enc, b_enc2d, b_dec2d):
    BH = 2048
    return pl.pallas_call(
        _enc_body,
        grid=(HID // BH,),
        in_specs=[
            pl.BlockSpec((B, D_IN), lambda j: (0, 0)),
            pl.BlockSpec((BH, D_IN), lambda j: (j, 0)),
            pl.BlockSpec((1, BH), lambda j: (0, j)),
            pl.BlockSpec((1, D_IN), lambda j: (0, 0)),
        ],
        out_specs=pl.BlockSpec((B, BH), lambda j: (0, j)),
        out_shape=jax.ShapeDtypeStruct((B, HID), jnp.float32),
    )(x, W_enc, b_enc2d, b_dec2d)


def _sc_body(pre_hbm, wdec_hbm, bdec_hbm, out_hbm,
             row_v, cand_v, candi_v, topv_v, topi_v, topi32_v,
             eqi_v, bc_v, rows_v, acc_v, bdec_v, sem):
    wid = lax.axis_index("s") * NSC + lax.axis_index("c")
    lane = lax.iota(jnp.int32, L)
    ones_i = jnp.ones((L,), jnp.int32)
    zeros_i = jnp.zeros((L,), jnp.int32)
    bigi = jnp.full((L,), jnp.int32(1 << 30), jnp.int32)
    lane0 = lane == 0

    pltpu.sync_copy(bdec_hbm, bdec_v)

    for r in range(2):
        row_id = wid * 2 + r
        pltpu.sync_copy(pre_hbm.at[row_id], row_v)

        neg1f = jnp.full((L,), -1.0, jnp.float32)

        def pbody(jj, hs):
            h1, h2 = hs
            for u in range(4):
                j = jj * 4 + u
                v = row_v[pl.ds(j * L, L)]
                h2 = jnp.maximum(h2, jnp.minimum(h1, v))
                h1 = jnp.maximum(h1, v)
            return (h1, h2)
        h1, h2 = lax.fori_loop(0, CHUNKS // 4, pbody, (neg1f, neg1f))
        t0vec = jnp.full((L,), jnp.min(h2), jnp.float32)
        tvec = t0vec
        lo0 = plsc.bitcast(t0vec, jnp.int32)[0]
        hi0 = plsc.bitcast(jnp.full((L,), jnp.max(h1), jnp.float32),
                           jnp.int32)[0]

        neg1 = jnp.full((L,), -1.0, jnp.float32)

        def cinit(i, _):
            for u in range(4):
                cand_v[pl.ds((i * 4 + u) * L, L)] = neg1
            return 0
        lax.fori_loop(0, SLOT // 4, cinit, 0)

        def sbody(jj, cnt_v):
            for u in range(2):
                j = jj * 2 + u
                v = row_v[pl.ds(j * L, L)]
                m = v >= tvec
                addr = jnp.minimum(cnt_v, SLOT - 1) * L + lane
                plsc.store_scatter(cand_v, [addr], v, mask=m)
                plsc.store_scatter(candi_v, [addr], j * L + lane, mask=m)
                cnt_v = cnt_v + jnp.where(m, 1, 0)
            return cnt_v
        lax.fori_loop(0, CHUNKS // 2, sbody, zeros_i)

        def bsbody(it, st):
            lo, hi = st
            mid = lo + ((hi - lo + 1) >> 1)
            tm = plsc.bitcast(jnp.full((L,), mid, jnp.int32), jnp.float32)

            def cb(i, cv):
                return cv + jnp.where(cand_v[pl.ds(i * L, L)] >= tm, 1, 0)
            c = jnp.sum(lax.fori_loop(0, SLOT, cb, zeros_i))
            ge = c >= K
            return (jnp.where(ge, mid, lo), jnp.where(ge, hi, mid - 1))
        tstar, _ = lax.fori_loop(0, 31, bsbody, (lo0, hi0))
        tsv = plsc.bitcast(jnp.full((L,), tstar, jnp.int32), jnp.float32)

        def gcomp(i, cnt):
            cv_ = cand_v[pl.ds(i * L, L)]
            ci_ = candi_v[pl.ds(i * L, L)]
            m = cv_ > tsv
            off = jnp.minimum(cnt, K)
            plsc.store_compressed(topv_v.at[pl.ds(off, L)], cv_, mask=m)
            plsc.store_compressed(topi_v.at[pl.ds(off, L)], ci_, mask=m)
            return cnt + plsc.all_reduce_population_count(m)[0]
        cnt_gt = jnp.minimum(lax.fori_loop(0, SLOT, gcomp, jnp.int32(0)),
                             jnp.int32(K - 1))

        for i in range(EQN):
            eqi_v[pl.ds(i * L, L)] = bigi

        def ecomp(i, cnt):
            cv_ = cand_v[pl.ds(i * L, L)]
            ci_ = candi_v[pl.ds(i * L, L)]
            m = cv_ == tsv
            off = jnp.minimum(cnt, (EQN - 1) * L)
            plsc.store_compressed(eqi_v.at[pl.ds(off, L)], ci_, mask=m)
            return cnt + plsc.all_reduce_population_count(m)[0]
        lax.fori_loop(0, SLOT, ecomp, jnp.int32(0))

        def rbody(rr, _):
            def mb(i, mv):
                return jnp.minimum(mv, eqi_v[pl.ds(i * L, L)])
            mi = jnp.min(lax.fori_loop(0, EQN, mb, bigi))
            mis = jnp.full((L,), mi, jnp.int32)

            def rm(i, _):
                ci_ = eqi_v[pl.ds(i * L, L)]
                eqi_v[pl.ds(i * L, L)] = jnp.where(ci_ == mis, bigi, ci_)
                return 0
            lax.fori_loop(0, EQN, rm, 0)
            ks = jnp.full((L,), cnt_gt + rr, jnp.int32)
            plsc.store_scatter(topi_v, [ks],
                               jnp.minimum(mis, jnp.int32(HID - 1)),
                               mask=lane0)
            plsc.store_scatter(topv_v, [ks], tsv, mask=lane0)
            return 0
        lax.fori_loop(0, K - cnt_gt, rbody, 0)

        for g in range(2):
            topi32_v[pl.ds(g * L, L)] = topi_v[pl.ds(g * L, L)]
            va = topv_v[pl.ds(g * L, L)]
            for j in range(L):
                addr = lane * L + g * (L * L) + ((lane + j) & (L - 1))
                plsc.store_scatter(bc_v, [addr], va)

        pltpu.async_copy(wdec_hbm.at[topi32_v], rows_v, sem).wait()

        def abody(c, _):
            acc_v[pl.ds(c * L, L)] = bdec_v[pl.ds(c * L, L)]
            return 0
        lax.fori_loop(0, D_IN // L, abody, 0)

        for kb in range(K // 8):
            bcs = [bc_v[pl.ds((kb * 8 + u) * L, L)] for u in range(8)]

            def dbody(c, _, kb=kb, bcs=bcs):
                a = acc_v[pl.ds(c * L, L)]
                for u in range(8):
                    a = a + bcs[u] * rows_v[kb * 8 + u, pl.ds(c * L, L)]
                acc_v[pl.ds(c * L, L)] = a
                return 0
            lax.fori_loop(0, D_IN // L, dbody, 0)

        pltpu.sync_copy(acc_v, out_hbm.at[row_id])


@functools.partial(
    pl.kernel,
    out_type=jax.ShapeDtypeStruct((B, D_IN), jnp.float32),
    mesh=plsc.VectorSubcoreMesh(core_axis_name="c", subcore_axis_name="s",
                                num_cores=NSC, num_subcores=NSUB),
    compiler_params=pltpu.CompilerParams(needs_layout_passes=False),
    scratch_types=[
        pltpu.VMEM((HID,), jnp.float32),
        pltpu.VMEM((SLOT * L,), jnp.float32),
        pltpu.VMEM((SLOT * L,), jnp.int32),
        pltpu.VMEM((K + L,), jnp.float32),
        pltpu.VMEM((K + L,), jnp.int32),
        pltpu.VMEM((K,), jnp.int32),
        pltpu.VMEM((EQN * L,), jnp.int32),
        pltpu.VMEM((K * L,), jnp.float32),
        pltpu.VMEM((K, D_IN), jnp.float32),
        pltpu.VMEM((D_IN,), jnp.float32),
        pltpu.VMEM((D_IN,), jnp.float32),
        pltpu.SemaphoreType.DMA,
    ],
)
def _sc_topk_decode(pre_hbm, wdec_hbm, bdec_hbm, out_hbm, *scratch):
    _sc_body(pre_hbm, wdec_hbm, bdec_hbm, out_hbm, *scratch)


def kernel(x, W_enc, b_enc, W_dec, b_dec):
    pre = _encode(x, W_enc, b_enc.reshape(1, HID), b_dec.reshape(1, D_IN))
    return _sc_topk_decode(pre, W_dec, b_dec)


# speedup vs baseline: 2.7636x; 1.0620x over previous
import functools

import jax
import jax.numpy as jnp
from jax import lax
from jax.experimental import pallas as pl
from jax.experimental.pallas import tpu as pltpu
from jax.experimental.pallas import tpu_sc as plsc

D_IN = 1024
HID = 32768
K = 32
B = 64
L = 16
CHUNKS = HID // L
NBUCKET = 1024
BSHIFT = 21
SLOT = 32
EQN = 4
NSC = 2
NSUB = 16


def _enc_body(x_ref, w_ref, benc_ref, bdec_ref, o_ref):
    sae = x_ref[...] - bdec_ref[...]
    acts = lax.dot_general(sae, w_ref[...], (((1,), (1,)), ((), ())),
                           preferred_element_type=jnp.float32)
    o_ref[...] = jnp.maximum(acts + benc_ref[...], 0.0)


def _encode(x, W_enc, b_enc2d, b_dec2d):
    BH = 2048
    return pl.pallas_call(
        _enc_body,
        grid=(HID // BH,),
        in_specs=[
            pl.BlockSpec((B, D_IN), lambda j: (0, 0)),
            pl.BlockSpec((BH, D_IN), lambda j: (j, 0)),
            pl.BlockSpec((1, BH), lambda j: (0, j)),
            pl.BlockSpec((1, D_IN), lambda j: (0, 0)),
        ],
        out_specs=pl.BlockSpec((B, BH), lambda j: (0, j)),
        out_shape=jax.ShapeDtypeStruct((B, HID), jnp.float32),
    )(x, W_enc, b_enc2d, b_dec2d)


def _sc_body(pre_hbm, wdec_hbm, bdec_hbm, out_hbm,
             row_v, gmax_v, cand_v, candi_v, topv_v, topi_v, topi32_v,
             eqi_v, bc_v, rows_v, acc_v, bdec_v, sem):
    wid = lax.axis_index("s") * NSC + lax.axis_index("c")
    lane = lax.iota(jnp.int32, L)
    ones_i = jnp.ones((L,), jnp.int32)
    zeros_i = jnp.zeros((L,), jnp.int32)
    bigi = jnp.full((L,), jnp.int32(1 << 30), jnp.int32)
    lane0 = lane == 0

    pltpu.sync_copy(bdec_hbm, bdec_v)

    for r in range(2):
        row_id = wid * 2 + r
        pltpu.sync_copy(pre_hbm.at[row_id], row_v)

        neg1f = jnp.full((L,), -1.0, jnp.float32)
        NGRP = CHUNKS // 16

        def pbody(g, hs):
            h1, h2 = hs
            gm = neg1f
            for u in range(16):
                v = row_v[pl.ds((g * 16 + u) * L, L)]
                h2 = jnp.maximum(h2, jnp.minimum(h1, v))
                h1 = jnp.maximum(h1, v)
                gm = jnp.maximum(gm, v)
            gmax_v[pl.ds(g * L, L)] = gm
            return (h1, h2)
        h1, h2 = lax.fori_loop(0, NGRP, pbody, (neg1f, neg1f))
        t0vec = jnp.full((L,), jnp.min(h2), jnp.float32)
        tvec = t0vec
        lo0 = plsc.bitcast(t0vec, jnp.int32)[0]
        hi0 = plsc.bitcast(jnp.full((L,), jnp.max(h1), jnp.float32),
                           jnp.int32)[0]

        neg1 = jnp.full((L,), -1.0, jnp.float32)

        def cinit(i, _):
            for u in range(4):
                cand_v[pl.ds((i * 4 + u) * L, L)] = neg1
            return 0
        lax.fori_loop(0, SLOT // 4, cinit, 0)

        def sbody(g, cnt_v):
            gmv = gmax_v[pl.ds(g * L, L)]
            hit = plsc.all_reduce_population_count(gmv >= tvec)[0]

            def scan(cv):
                for u in range(16):
                    j = g * 16 + u
                    v = row_v[pl.ds(j * L, L)]
                    m = v >= tvec
                    addr = jnp.minimum(cv, SLOT - 1) * L + lane
                    plsc.store_scatter(cand_v, [addr], v, mask=m)
                    plsc.store_scatter(candi_v, [addr], j * L + lane,
                                       mask=m)
                    cv = cv + jnp.where(m, 1, 0)
                return cv
            return lax.cond(hit > 0, scan, lambda cv: cv, cnt_v)
        lax.fori_loop(0, NGRP, sbody, zeros_i)

        def bsbody(it, st):
            lo, hi = st
            mid = lo + ((hi - lo + 1) >> 1)
            tm = plsc.bitcast(jnp.full((L,), mid, jnp.int32), jnp.float32)

            def cb(i, cv):
                return cv + jnp.where(cand_v[pl.ds(i * L, L)] >= tm, 1, 0)
            c = jnp.sum(lax.fori_loop(0, SLOT, cb, zeros_i))
            ge = c >= K
            return (jnp.where(ge, mid, lo), jnp.where(ge, hi, mid - 1))
        tstar, _ = lax.fori_loop(0, 31, bsbody, (lo0, hi0))
        tsv = plsc.bitcast(jnp.full((L,), tstar, jnp.int32), jnp.float32)

        def gcomp(i, cnt):
            cv_ = cand_v[pl.ds(i * L, L)]
            ci_ = candi_v[pl.ds(i * L, L)]
            m = cv_ > tsv
            off = jnp.minimum(cnt, K)
            plsc.store_compressed(topv_v.at[pl.ds(off, L)], cv_, mask=m)
            plsc.store_compressed(topi_v.at[pl.ds(off, L)], ci_, mask=m)
            return cnt + plsc.all_reduce_population_count(m)[0]
        cnt_gt = jnp.minimum(lax.fori_loop(0, SLOT, gcomp, jnp.int32(0)),
                             jnp.int32(K - 1))

        for i in range(EQN):
            eqi_v[pl.ds(i * L, L)] = bigi

        def ecomp(i, cnt):
            cv_ = cand_v[pl.ds(i * L, L)]
            ci_ = candi_v[pl.ds(i * L, L)]
            m = cv_ == tsv
            off = jnp.minimum(cnt, (EQN - 1) * L)
            plsc.store_compressed(eqi_v.at[pl.ds(off, L)], ci_, mask=m)
            return cnt + plsc.all_reduce_population_count(m)[0]
        lax.fori_loop(0, SLOT, ecomp, jnp.int32(0))

        def rbody(rr, _):
            def mb(i, mv):
                return jnp.minimum(mv, eqi_v[pl.ds(i * L, L)])
            mi = jnp.min(lax.fori_loop(0, EQN, mb, bigi))
            mis = jnp.full((L,), mi, jnp.int32)

            def rm(i, _):
                ci_ = eqi_v[pl.ds(i * L, L)]
                eqi_v[pl.ds(i * L, L)] = jnp.where(ci_ == mis, bigi, ci_)
                return 0
            lax.fori_loop(0, EQN, rm, 0)
            ks = jnp.full((L,), cnt_gt + rr, jnp.int32)
            plsc.store_scatter(topi_v, [ks],
                               jnp.minimum(mis, jnp.int32(HID - 1)),
                               mask=lane0)
            plsc.store_scatter(topv_v, [ks], tsv, mask=lane0)
            return 0
        lax.fori_loop(0, K - cnt_gt, rbody, 0)

        for g in range(2):
            topi32_v[pl.ds(g * L, L)] = topi_v[pl.ds(g * L, L)]
            va = topv_v[pl.ds(g * L, L)]
            for j in range(L):
                addr = lane * L + g * (L * L) + ((lane + j) & (L - 1))
                plsc.store_scatter(bc_v, [addr], va)

        pltpu.async_copy(wdec_hbm.at[topi32_v], rows_v, sem).wait()

        def abody(c, _):
            acc_v[pl.ds(c * L, L)] = bdec_v[pl.ds(c * L, L)]
            return 0
        lax.fori_loop(0, D_IN // L, abody, 0)

        for kb in range(K // 8):
            bcs = [bc_v[pl.ds((kb * 8 + u) * L, L)] for u in range(8)]

            def dbody(c, _, kb=kb, bcs=bcs):
                a = acc_v[pl.ds(c * L, L)]
                for u in range(8):
                    a = a + bcs[u] * rows_v[kb * 8 + u, pl.ds(c * L, L)]
                acc_v[pl.ds(c * L, L)] = a
                return 0
            lax.fori_loop(0, D_IN // L, dbody, 0)

        pltpu.sync_copy(acc_v, out_hbm.at[row_id])


@functools.partial(
    pl.kernel,
    out_type=jax.ShapeDtypeStruct((B, D_IN), jnp.float32),
    mesh=plsc.VectorSubcoreMesh(core_axis_name="c", subcore_axis_name="s",
                                num_cores=NSC, num_subcores=NSUB),
    compiler_params=pltpu.CompilerParams(needs_layout_passes=False),
    scratch_types=[
        pltpu.VMEM((HID,), jnp.float32),
        pltpu.VMEM((CHUNKS,), jnp.float32),
        pltpu.VMEM((SLOT * L,), jnp.float32),
        pltpu.VMEM((SLOT * L,), jnp.int32),
        pltpu.VMEM((K + L,), jnp.float32),
        pltpu.VMEM((K + L,), jnp.int32),
        pltpu.VMEM((K,), jnp.int32),
        pltpu.VMEM((EQN * L,), jnp.int32),
        pltpu.VMEM((K * L,), jnp.float32),
        pltpu.VMEM((K, D_IN), jnp.float32),
        pltpu.VMEM((D_IN,), jnp.float32),
        pltpu.VMEM((D_IN,), jnp.float32),
        pltpu.SemaphoreType.DMA,
    ],
)
def _sc_topk_decode(pre_hbm, wdec_hbm, bdec_hbm, out_hbm, *scratch):
    _sc_body(pre_hbm, wdec_hbm, bdec_hbm, out_hbm, *scratch)


def kernel(x, W_enc, b_enc, W_dec, b_dec):
    pre = _encode(x, W_enc, b_enc.reshape(1, HID), b_dec.reshape(1, D_IN))
    return _sc_topk_decode(pre, W_dec, b_dec)
